# baseline (speedup 1.0000x reference)
import functools

import jax
import jax.numpy as jnp
from jax import lax
from jax.experimental import pallas as pl
from jax.experimental.pallas import tpu as pltpu
from jax.experimental.pallas import tpu_sc as plsc

N_NODES = 10000
DIM = 128

NC = 2
NS = 16
NW = NC * NS

B = 4096
L = 50
N = B * L
PER_W = N // NW
G = 128
NBUF = 5
STEPS = PER_W // (G * NBUF)


@functools.cache
def _make_sc_gather():
    mesh = plsc.VectorSubcoreMesh(
        core_axis_name="c", subcore_axis_name="s", num_cores=NC, num_subcores=NS
    )

    @functools.partial(
        pl.kernel,
        out_type=jax.ShapeDtypeStruct((N, DIM), jnp.float32),
        mesh=mesh,
        scratch_types=[
            pltpu.VMEM((PER_W,), jnp.int32),
            [pltpu.VMEM((G, DIM), jnp.float32)] * NBUF,
            [pltpu.SemaphoreType.DMA] * NBUF,
            [pltpu.SemaphoreType.DMA] * NBUF,
        ],
        compiler_params=pltpu.CompilerParams(use_tc_tiling_on_sc=True),
    )
    def k(idx_hbm, table_hbm, out2d, idx_v, bufs, gsems, wsems):
        wid = lax.axis_index("s") * NC + lax.axis_index("c")

        pltpu.sync_copy(idx_hbm.at[pl.ds(wid * PER_W, PER_W)], idx_v)
        base = wid * PER_W

        @pl.loop(0, STEPS)
        def _step(step):
            for sl in range(NBUF):
                @pl.when(step > 0)
                def _wait_write(sl=sl):
                    pltpu.make_async_copy(
                        bufs[sl], out2d.at[pl.ds(base, G)], wsems[sl]
                    ).wait()

                pltpu.async_copy(
                    table_hbm.at[idx_v.at[pl.ds((step * NBUF + sl) * G, G)]],
                    bufs[sl],
                    gsems[sl],
                )
            for sl in range(NBUF):
                g = step * NBUF + sl
                pltpu.make_async_copy(
                    table_hbm.at[idx_v.at[pl.ds(g * G, G)]], bufs[sl], gsems[sl]
                ).wait()
                pltpu.async_copy(
                    bufs[sl], out2d.at[pl.ds(base + g * G, G)], wsems[sl]
                )

        for sl in range(NBUF):
            pltpu.make_async_copy(
                bufs[sl], out2d.at[pl.ds(base, G)], wsems[sl]
            ).wait()

    return k


@jax.jit
def kernel(indices, table):
    idx_t = indices.astype(jnp.int32).T.reshape(-1)
    out_t = _make_sc_gather()(idx_t, table)
    return out_t.reshape(L, B, DIM).transpose(1, 0, 2)


# speedup vs baseline: 1.4488x; 1.4488x over previous
import functools

import jax
import jax.numpy as jnp
from jax import lax
from jax.experimental import pallas as pl
from jax.experimental.pallas import tpu as pltpu
from jax.experimental.pallas import tpu_sc as plsc

N_NODES = 10000
DIM = 128

NC = 2
NS = 16
NW = NC * NS

B = 4096
L = 50
N = B * L
PER_W = N // NW
G = 80
NBUF = 4
STEPS = PER_W // (G * NBUF)


@functools.cache
def _make_sc_gather():
    mesh = plsc.VectorSubcoreMesh(
        core_axis_name="c", subcore_axis_name="s", num_cores=NC, num_subcores=NS
    )

    @functools.partial(
        pl.kernel,
        out_type=jax.ShapeDtypeStruct((N, DIM), jnp.float32),
        mesh=mesh,
        scratch_types=[
            pltpu.VMEM((PER_W,), jnp.int32),
            [pltpu.VMEM((G, DIM), jnp.float32)] * NBUF,
            pltpu.VMEM_SHARED((N_NODES, DIM), jnp.float32),
            [pltpu.SemaphoreType.DMA] * NBUF,
            [pltpu.SemaphoreType.DMA] * NBUF,
        ],
        compiler_params=pltpu.CompilerParams(use_tc_tiling_on_sc=True),
    )
    def k(idx_hbm, table_hbm, out2d, idx_v, bufs, table_sh, gsems, wsems):
        wid = lax.axis_index("s") * NC + lax.axis_index("c")

        sid = lax.axis_index("s")

        @pl.when(sid < 10)
        def _stage():
            pltpu.sync_copy(
                table_hbm.at[pl.ds(sid * 1000, 1000)],
                table_sh.at[pl.ds(sid * 1000, 1000)],
            )

        pltpu.sync_copy(idx_hbm.at[pl.ds(wid * PER_W, PER_W)], idx_v)
        plsc.subcore_barrier()
        base = wid * PER_W

        @pl.loop(0, STEPS)
        def _step(step):
            for sl in range(NBUF):
                @pl.when(step > 0)
                def _wait_write(sl=sl):
                    pltpu.make_async_copy(
                        bufs[sl], out2d.at[pl.ds(base, G)], wsems[sl]
                    ).wait()

                pltpu.async_copy(
                    table_sh.at[idx_v.at[pl.ds((step * NBUF + sl) * G, G)]],
                    bufs[sl],
                    gsems[sl],
                )
            for sl in range(NBUF):
                g = step * NBUF + sl
                pltpu.make_async_copy(
                    table_sh.at[idx_v.at[pl.ds(g * G, G)]], bufs[sl], gsems[sl]
                ).wait()
                pltpu.async_copy(
                    bufs[sl], out2d.at[pl.ds(base + g * G, G)], wsems[sl]
                )

        for sl in range(NBUF):
            pltpu.make_async_copy(
                bufs[sl], out2d.at[pl.ds(base, G)], wsems[sl]
            ).wait()

    return k


@jax.jit
def kernel(indices, table):
    idx_t = indices.astype(jnp.int32).T.reshape(-1)
    out_t = _make_sc_gather()(idx_t, table)
    return out_t.reshape(L, B, DIM).transpose(1, 0, 2)
